# baseline (speedup 1.0000x reference)
import functools
import math

import jax
import jax.numpy as jnp
from jax import lax
from jax.experimental import pallas as pl
from jax.experimental.pallas import tpu as pltpu
from jax.experimental.pallas import tpu_sc as plsc

EMB_DIM = 64
LANES = 16
CHUNK = 512


def _emb_kernel(b_per_w, n_chunks, nc, x_hbm, table_hbm, out_hbm,
                idx_v, rows_v, sem):
    wid = lax.axis_index("s") * nc + lax.axis_index("c")
    base = wid * b_per_w
    pltpu.sync_copy(x_hbm.at[pl.ds(base, b_per_w)], idx_v)

    def step(g, _):
        off = g * CHUNK
        pltpu.async_copy(table_hbm.at[idx_v.at[pl.ds(off, CHUNK)]],
                         rows_v, sem).wait()

        def scale_row(i, _):
            for k in range(EMB_DIM // LANES):
                sl = pl.ds(k * LANES, LANES)
                rows_v[i, sl] = rows_v[i, sl] * 8.0
            return 0

        lax.fori_loop(0, CHUNK, scale_row, 0)
        pltpu.sync_copy(rows_v, out_hbm.at[pl.ds(base + off, CHUNK)])
        return 0

    lax.fori_loop(0, n_chunks, step, 0)


def kernel(x, table):
    batch, hist = x.shape
    vocab, dim = table.shape
    assert dim == EMB_DIM
    n = batch * hist
    info = plsc.get_sparse_core_info()
    nc, ns = info.num_cores, info.num_subcores
    nw = nc * ns
    b_per_w = n // nw
    assert b_per_w * nw == n and b_per_w % CHUNK == 0
    n_chunks = b_per_w // CHUNK

    xf = x.reshape(n).astype(jnp.int32)
    mesh = plsc.VectorSubcoreMesh(core_axis_name="c", subcore_axis_name="s")
    run = pl.kernel(
        functools.partial(_emb_kernel, b_per_w, n_chunks, nc),
        mesh=mesh,
        compiler_params=pltpu.CompilerParams(use_tc_tiling_on_sc=False),
        out_type=jax.ShapeDtypeStruct((n, dim), jnp.float32),
        scratch_types=[
            pltpu.VMEM((b_per_w,), jnp.int32),
            pltpu.VMEM((CHUNK, dim), jnp.float32),
            pltpu.SemaphoreType.DMA,
        ],
    )
    out = run(xf, table)
    return out.reshape(batch, hist, dim)


# speedup vs baseline: 1.0781x; 1.0781x over previous
import functools
import math

import jax
import jax.numpy as jnp
from jax import lax
from jax.experimental import pallas as pl
from jax.experimental.pallas import tpu as pltpu
from jax.experimental.pallas import tpu_sc as plsc

EMB_DIM = 64
LANES = 16


def _emb_kernel(rows_pw, hist, nc, x_hbm, table_hbm, out_hbm,
                idx_v, buf_a, buf_b, sem_a, sem_b):
    wid = lax.axis_index("s") * nc + lax.axis_index("c")
    row0 = wid * rows_pw
    pltpu.sync_copy(x_hbm.at[pl.ds(row0, rows_pw)], idx_v)

    def gather(c, buf, sem):
        return pltpu.make_async_copy(table_hbm.at[idx_v.at[c]], buf, sem)

    def process(c, buf):
        def scale_row(i, _):
            for k in range(EMB_DIM // LANES):
                sl = pl.ds(k * LANES, LANES)
                buf[i, sl] = buf[i, sl] * 8.0
            return 0
        lax.fori_loop(0, hist, scale_row, 0)
        pltpu.sync_copy(buf, out_hbm.at[row0 + c])

    gather(0, buf_a, sem_a).start()

    def step(s, _):
        c0 = 2 * s
        gather(c0 + 1, buf_b, sem_b).start()
        gather(c0, buf_a, sem_a).wait()
        process(c0, buf_a)

        @pl.when(s < rows_pw // 2 - 1)
        def _():
            gather(c0 + 2, buf_a, sem_a).start()

        gather(c0 + 1, buf_b, sem_b).wait()
        process(c0 + 1, buf_b)
        return 0

    lax.fori_loop(0, rows_pw // 2, step, 0)


def kernel(x, table):
    batch, hist = x.shape
    vocab, dim = table.shape
    assert dim == EMB_DIM
    info = plsc.get_sparse_core_info()
    nc, ns = info.num_cores, info.num_subcores
    nw = nc * ns
    rows_pw = batch // nw
    assert rows_pw * nw == batch and rows_pw % 2 == 0

    mesh = plsc.VectorSubcoreMesh(core_axis_name="c", subcore_axis_name="s")
    run = pl.kernel(
        functools.partial(_emb_kernel, rows_pw, hist, nc),
        mesh=mesh,
        compiler_params=pltpu.CompilerParams(use_tc_tiling_on_sc=False),
        out_type=jax.ShapeDtypeStruct((batch, hist, dim), jnp.float32),
        scratch_types=[
            pltpu.VMEM((rows_pw, hist), jnp.int32),
            pltpu.VMEM((hist, dim), jnp.float32),
            pltpu.VMEM((hist, dim), jnp.float32),
            pltpu.SemaphoreType.DMA,
            pltpu.SemaphoreType.DMA,
        ],
    )
    return run(x, table)
